# baseline (speedup 1.0000x reference)
import functools

import jax
import jax.numpy as jnp
from jax import lax
from jax.experimental import pallas as pl
from jax.experimental.pallas import tpu as pltpu
from jax.experimental.pallas import tpu_sc as plsc

B = 16
N = 2048
K = 16384
L = 16
BLK = 128
NBLK = N // BLK
K_HALF = K // 2


def _diag_body(t_hbm, out_hbm, buf, diag_v, sem_in, sem_out):
    in_cps = []
    for j in range(NBLK):
        cp = pltpu.make_async_copy(
            t_hbm.at[:, pl.ds(BLK * j, BLK), pl.ds(BLK * j, BLK)],
            buf.at[j], sem_in)
        cp.start()
        in_cps.append(cp)

    ii = lax.broadcasted_iota(jnp.int32, (BLK, BLK), 0)
    jj = lax.broadcasted_iota(jnp.int32, (BLK, BLK), 1)
    eq = (ii == jj)[None]
    for j in range(NBLK):
        in_cps[j].wait()
        blk = buf[j]
        diag_v[:, pl.ds(BLK * j, BLK)] = jnp.sum(jnp.where(eq, blk, 0.0),
                                                 axis=1)

    out_cps = []
    for b in range(B):
        cp = pltpu.make_async_copy(diag_v.at[b], out_hbm.at[pl.ds(N * b, N)],
                                   sem_out)
        cp.start()
        out_cps.append(cp)
    for cp in out_cps:
        cp.wait()


_diag_extract = pl.pallas_call(
    _diag_body,
    in_specs=[pl.BlockSpec(memory_space=pl.ANY)],
    out_specs=pl.BlockSpec(memory_space=pl.ANY),
    out_shape=jax.ShapeDtypeStruct((B * N,), jnp.float32),
    scratch_shapes=[
        pltpu.VMEM((NBLK, B, BLK, BLK), jnp.float32),
        pltpu.VMEM((B, N), jnp.float32),
        pltpu.SemaphoreType.DMA,
        pltpu.SemaphoreType.DMA,
    ],
)


def _sc_body(diag_hbm, idx_hbm, out_hbm, diag_v, idx_v, out_v):
    c = lax.axis_index("c")
    s = lax.axis_index("s")

    pltpu.sync_copy(diag_hbm.at[pl.ds(s * N, N)], diag_v)
    base = c * K_HALF
    pltpu.sync_copy(idx_hbm.at[pl.ds(base, K_HALF)], idx_v)

    def gat(g, carry):
        o = g * L
        iv = idx_v[pl.ds(o, L)]
        out_v[pl.ds(o, L)] = plsc.load_gather(diag_v, [iv])
        return carry
    lax.fori_loop(0, K_HALF // L, gat, 0)

    pltpu.sync_copy(out_v, out_hbm.at[s, pl.ds(base, K_HALF)])


_sc_lookup = functools.partial(
    pl.kernel,
    out_type=jax.ShapeDtypeStruct((B, K), jnp.float32),
    mesh=plsc.VectorSubcoreMesh(core_axis_name="c", subcore_axis_name="s"),
    compiler_params=pltpu.CompilerParams(needs_layout_passes=False),
    scratch_types=[
        pltpu.VMEM((N,), jnp.float32),
        pltpu.VMEM((K_HALF,), jnp.int32),
        pltpu.VMEM((K_HALF,), jnp.float32),
    ],
)(_sc_body)


def kernel(t, idx):
    diag = _diag_extract(t)
    return _sc_lookup(diag, idx.astype(jnp.int32))


# speedup vs baseline: 1.0932x; 1.0932x over previous
import functools

import jax
import jax.numpy as jnp
from jax import lax
from jax.experimental import pallas as pl
from jax.experimental.pallas import tpu as pltpu
from jax.experimental.pallas import tpu_sc as plsc

B = 16
N = 2048
K = 16384
L = 16
BLK = 128
NBLK = N // BLK
K_HALF = K // 2


def _diag_body(t_hbm, out_hbm, buf, diag_v, sem_in, sem_out):
    in_cps = []
    for j in range(NBLK):
        cp = pltpu.make_async_copy(
            t_hbm.at[:, pl.ds(BLK * j, BLK), pl.ds(BLK * j, BLK)],
            buf.at[j], sem_in)
        cp.start()
        in_cps.append(cp)

    ii = lax.broadcasted_iota(jnp.int32, (BLK, BLK), 0)
    jj = lax.broadcasted_iota(jnp.int32, (BLK, BLK), 1)
    eq = (ii == jj)[None]
    for j in range(NBLK):
        in_cps[j].wait()
        blk = buf[j]
        diag_v[:, pl.ds(BLK * j, BLK)] = jnp.sum(jnp.where(eq, blk, 0.0),
                                                 axis=1)

    out_cps = []
    for b in range(B):
        cp = pltpu.make_async_copy(diag_v.at[b], out_hbm.at[pl.ds(N * b, N)],
                                   sem_out)
        cp.start()
        out_cps.append(cp)
    for cp in out_cps:
        cp.wait()


_diag_extract = pl.pallas_call(
    _diag_body,
    in_specs=[pl.BlockSpec(memory_space=pl.ANY)],
    out_specs=pl.BlockSpec(memory_space=pl.ANY),
    out_shape=jax.ShapeDtypeStruct((B * N,), jnp.float32),
    scratch_shapes=[
        pltpu.VMEM((NBLK, B, BLK, BLK), jnp.float32),
        pltpu.VMEM((B, N), jnp.float32),
        pltpu.SemaphoreType.DMA,
        pltpu.SemaphoreType.DMA,
    ],
)


def _sc_body(diag_hbm, idx_hbm, out_hbm, diag_v, idx_v, out_v):
    c = lax.axis_index("c")
    s = lax.axis_index("s")

    pltpu.sync_copy(diag_hbm.at[pl.ds(s * N, N)], diag_v)
    base = c * K_HALF
    pltpu.sync_copy(idx_hbm.at[pl.ds(base, K_HALF)], idx_v)

    @plsc.parallel_loop(0, K_HALF // L, unroll=8)
    def gat(g):
        o = g * L
        iv = idx_v[pl.ds(o, L)]
        out_v[pl.ds(o, L)] = plsc.load_gather(diag_v, [iv])

    pltpu.sync_copy(out_v, out_hbm.at[s, pl.ds(base, K_HALF)])


_sc_lookup = functools.partial(
    pl.kernel,
    out_type=jax.ShapeDtypeStruct((B, K), jnp.float32),
    mesh=plsc.VectorSubcoreMesh(core_axis_name="c", subcore_axis_name="s"),
    compiler_params=pltpu.CompilerParams(needs_layout_passes=False),
    scratch_types=[
        pltpu.VMEM((N,), jnp.float32),
        pltpu.VMEM((K_HALF,), jnp.int32),
        pltpu.VMEM((K_HALF,), jnp.float32),
    ],
)(_sc_body)


def kernel(t, idx):
    diag = _diag_extract(t)
    return _sc_lookup(diag, idx.astype(jnp.int32))
